# baseline (speedup 1.0000x reference)
import functools

import jax
import jax.numpy as jnp
from jax import lax
from jax.experimental import pallas as pl
from jax.experimental.pallas import tpu as pltpu
from jax.experimental.pallas import tpu_sc as plsc

_EPS = 1e-12
_TB = 2048
_NW = 32
_CHT = 4096


def _sc_gather_t(wt_t, ids):
    nfeat, vocab = wt_t.shape
    n = ids.shape[0]
    ncb = n // _CHT
    nk = (nfeat + _NW - 1) // _NW

    mesh = plsc.VectorSubcoreMesh(core_axis_name="c", subcore_axis_name="s")

    @functools.partial(
        pl.kernel,
        mesh=mesh,
        compiler_params=pltpu.CompilerParams(needs_layout_passes=False),
        out_type=jax.ShapeDtypeStruct((nfeat, n), jnp.float32),
        scratch_types=[
            pltpu.VMEM((vocab,), jnp.float32),
            pltpu.VMEM((_CHT,), jnp.int32),
            pltpu.VMEM((_CHT,), jnp.int32),
            pltpu.VMEM((_CHT,), jnp.float32),
            pltpu.VMEM((_CHT,), jnp.float32),
            pltpu.SemaphoreType.DMA,
            pltpu.SemaphoreType.DMA,
            pltpu.SemaphoreType.DMA,
            pltpu.SemaphoreType.DMA,
        ],
    )
    def k(wt_hbm, idx_hbm, out_hbm, row_v, ids0, ids1, out0, out1,
          isem0, isem1, osem0, osem1):
        wid = lax.axis_index("s") * 2 + lax.axis_index("c")
        idbuf = (ids0, ids1)
        isem = (isem0, isem1)
        obuf = (out0, out1)
        osem = (osem0, osem1)
        for kf in range(nk):
            f = wid + _NW * kf

            @pl.when(f < nfeat)
            def _():
                pltpu.sync_copy(wt_hbm.at[f], row_v)
                pending = pltpu.async_copy(
                    idx_hbm.at[pl.ds(0, _CHT)], idbuf[0], isem[0])
                odrain = [None, None]
                for cb in range(ncb):
                    cur = pending
                    if cb + 1 < ncb:
                        pending = pltpu.async_copy(
                            idx_hbm.at[pl.ds((cb + 1) * _CHT, _CHT)],
                            idbuf[(cb + 1) % 2], isem[(cb + 1) % 2])
                    cur.wait()
                    ids_v = idbuf[cb % 2]
                    out_v = obuf[cb % 2]
                    if odrain[cb % 2] is not None:
                        odrain[cb % 2].wait()

                    @plsc.parallel_loop(0, _CHT, step=16, unroll=8)
                    def _gather(i):
                        idx = ids_v[pl.ds(i, 16)]
                        out_v[pl.ds(i, 16)] = plsc.load_gather(row_v, [idx])

                    odrain[cb % 2] = pltpu.async_copy(
                        out_v, out_hbm.at[f, pl.ds(cb * _CHT, _CHT)],
                        osem[cb % 2])
                for dr in odrain:
                    if dr is not None:
                        dr.wait()

    return k(wt_t, ids)


def _ln(x, w, b, dim):
    u = jnp.sum(x, axis=-1, keepdims=True) * (1.0 / dim)
    s = jnp.sum(x * x, axis=-1, keepdims=True) * (1.0 / dim) - u * u
    return w * ((x - u) * lax.rsqrt(s + _EPS)) + b


def _tc_video_body(vf_ref, tt_ref, vidW_ref, tok_ref, pe_ref,
                   vcs, vc, vl2w, vl2b, out_ref):
    v = vf_ref[...]
    inv_v = 1.0 / v.shape[1]
    uv = jnp.sum(v, axis=-1, keepdims=True) * inv_v
    sv = jnp.sum(v * v, axis=-1, keepdims=True) * inv_v - uv * uv
    mv = jnp.dot(v.astype(jnp.bfloat16), vidW_ref[...],
                 preferred_element_type=jnp.float32)
    hv = (mv - uv * vcs[...]) * lax.rsqrt(sv + _EPS) + vc[...]
    hv = _ln(jnp.maximum(hv, 0.0), vl2w[...], vl2b[...], 768.0)
    ids = tt_ref[0]
    ids2 = ids.reshape(-1, 1)
    tt = jnp.where(ids2 == 0, tok_ref[0:1, :],
                   jnp.where(ids2 == 1, tok_ref[1:2, :], tok_ref[2:3, :]))
    out_ref[...] = (hv + tt + pe_ref[...]).astype(jnp.bfloat16)


def _tc_word_body(wet_ref, part_ref, wfcW_ref,
                  wcs, wc, wl2w, wl2b, flw, flb, out_ref):
    xt = wet_ref[...]
    inv = 1.0 / xt.shape[0]
    u_r = jnp.sum(xt, axis=0, keepdims=True) * inv
    s_r = jnp.sum(xt * xt, axis=0, keepdims=True) * inv - u_r * u_r
    u = u_r.reshape(-1, 1)
    s = s_r.reshape(-1, 1)
    mm = lax.dot_general(xt.astype(jnp.bfloat16), wfcW_ref[...],
                         (((0,), (0,)), ((), ())),
                         preferred_element_type=jnp.float32)
    hw = (mm - u * wcs[...]) * lax.rsqrt(s + _EPS) + wc[...]
    hw = _ln(jnp.maximum(hw, 0.0), wl2w[...], wl2b[...], 768.0)
    emb = hw + part_ref[...].astype(jnp.float32)
    out_ref[...] = _ln(emb, flw[...], flb[...], 768.0)


def _whole(i):
    return (0, 0)


def _tok_block(i):
    return (i, 0)


def _tc_video(vf, ttc, vidW, tok_table, pe_t, params):
    n, vid_feat = vf.shape
    hid = vidW.shape[1]
    in_specs = [
        pl.BlockSpec((_TB, vid_feat), _tok_block),
        pl.BlockSpec((1, 1, _TB), lambda i: (i, 0, 0)),
        pl.BlockSpec(vidW.shape, _whole),
        pl.BlockSpec(tok_table.shape, _whole),
        pl.BlockSpec(pe_t.shape, _whole),
    ] + [pl.BlockSpec(p.shape, _whole) for p in params]
    return pl.pallas_call(
        _tc_video_body,
        grid=(n // _TB,),
        in_specs=in_specs,
        out_specs=pl.BlockSpec((_TB, hid), _tok_block),
        out_shape=jax.ShapeDtypeStruct((n, hid), jnp.bfloat16),
    )(vf, ttc, vidW, tok_table, pe_t, *params)


def _tc_word(wet, partial, wfcW, params):
    wvec, n = wet.shape
    hid = wfcW.shape[1]
    in_specs = [
        pl.BlockSpec((wvec, _TB), lambda i: (0, i)),
        pl.BlockSpec((_TB, hid), _tok_block),
        pl.BlockSpec(wfcW.shape, _whole),
    ] + [pl.BlockSpec(p.shape, _whole) for p in params]
    return pl.pallas_call(
        _tc_word_body,
        grid=(n // _TB,),
        in_specs=in_specs,
        out_specs=pl.BlockSpec((_TB, hid), _tok_block),
        out_shape=jax.ShapeDtypeStruct((n, hid), jnp.float32),
    )(wet, partial, wfcW, *params)


def kernel(input_ids, video_features, token_type_ids, word_table, tok_table,
           wfc_ln1_w, wfc_ln1_b, wfc_W, wfc_b, wfc_ln2_w, wfc_ln2_b,
           vid_ln1_w, vid_ln1_b, vid_W, vid_b, vid_ln2_w, vid_ln2_b,
           final_ln_w, final_ln_b, pe):
    b, l = input_ids.shape
    n = b * l
    hid = vid_W.shape[1]

    ids = input_ids.reshape(n).astype(jnp.int32)
    wet = _sc_gather_t(word_table.T, ids)

    vf = video_features.reshape(n, video_features.shape[2])
    ttc = token_type_ids.astype(jnp.int32).reshape(n // _TB, 1, _TB)
    pe_t = jnp.tile(pe[:l], (_TB // l, 1))

    row = lambda p: p.reshape(1, -1)
    wfcWs = wfc_ln1_w[:, None] * wfc_W
    w_colsum = row(jnp.sum(wfcWs, axis=0))
    w_const = row(wfc_ln1_b @ wfc_W + wfc_b)
    vidWs = vid_ln1_w[:, None] * vid_W
    v_colsum = row(jnp.sum(vidWs, axis=0))
    v_const = row(vid_ln1_b @ vid_W + vid_b)

    partial = _tc_video(vf, ttc, vidWs.astype(jnp.bfloat16), tok_table, pe_t,
                        (v_colsum, v_const, row(vid_ln2_w), row(vid_ln2_b)))
    out = _tc_word(wet, partial, wfcWs.astype(jnp.bfloat16),
                   (w_colsum, w_const, row(wfc_ln2_w), row(wfc_ln2_b),
                    row(final_ln_w), row(final_ln_b)))
    return out.reshape(b, l, hid)


# speedup vs baseline: 1.0118x; 1.0118x over previous
import functools

import jax
import jax.numpy as jnp
from jax import lax
from jax.experimental import pallas as pl
from jax.experimental.pallas import tpu as pltpu
from jax.experimental.pallas import tpu_sc as plsc

_EPS = 1e-12
_TB = 1024
_NW = 32
_CHT = 4096


def _sc_gather_t(wt_t, ids):
    nfeat, vocab = wt_t.shape
    n = ids.shape[0]
    ncb = n // _CHT
    nk = (nfeat + _NW - 1) // _NW

    mesh = plsc.VectorSubcoreMesh(core_axis_name="c", subcore_axis_name="s")

    @functools.partial(
        pl.kernel,
        mesh=mesh,
        compiler_params=pltpu.CompilerParams(needs_layout_passes=False),
        out_type=jax.ShapeDtypeStruct((nfeat, n), jnp.float32),
        scratch_types=[
            pltpu.VMEM((vocab,), jnp.float32),
            pltpu.VMEM((_CHT,), jnp.int32),
            pltpu.VMEM((_CHT,), jnp.int32),
            pltpu.VMEM((_CHT,), jnp.float32),
            pltpu.VMEM((_CHT,), jnp.float32),
            pltpu.SemaphoreType.DMA,
            pltpu.SemaphoreType.DMA,
            pltpu.SemaphoreType.DMA,
            pltpu.SemaphoreType.DMA,
        ],
    )
    def k(wt_hbm, idx_hbm, out_hbm, row_v, ids0, ids1, out0, out1,
          isem0, isem1, osem0, osem1):
        wid = lax.axis_index("s") * 2 + lax.axis_index("c")
        idbuf = (ids0, ids1)
        isem = (isem0, isem1)
        obuf = (out0, out1)
        osem = (osem0, osem1)
        for kf in range(nk):
            f = wid + _NW * kf

            @pl.when(f < nfeat)
            def _():
                pltpu.sync_copy(wt_hbm.at[f], row_v)
                pending = pltpu.async_copy(
                    idx_hbm.at[pl.ds(0, _CHT)], idbuf[0], isem[0])
                odrain = [None, None]
                for cb in range(ncb):
                    cur = pending
                    if cb + 1 < ncb:
                        pending = pltpu.async_copy(
                            idx_hbm.at[pl.ds((cb + 1) * _CHT, _CHT)],
                            idbuf[(cb + 1) % 2], isem[(cb + 1) % 2])
                    cur.wait()
                    ids_v = idbuf[cb % 2]
                    out_v = obuf[cb % 2]
                    if odrain[cb % 2] is not None:
                        odrain[cb % 2].wait()

                    @plsc.parallel_loop(0, _CHT, step=16, unroll=8)
                    def _gather(i):
                        idx = ids_v[pl.ds(i, 16)]
                        out_v[pl.ds(i, 16)] = plsc.load_gather(row_v, [idx])

                    odrain[cb % 2] = pltpu.async_copy(
                        out_v, out_hbm.at[f, pl.ds(cb * _CHT, _CHT)],
                        osem[cb % 2])
                for dr in odrain:
                    if dr is not None:
                        dr.wait()

    return k(wt_t, ids)


def _ln(x, w, b, dim):
    u = jnp.sum(x, axis=-1, keepdims=True) * (1.0 / dim)
    s = jnp.sum(x * x, axis=-1, keepdims=True) * (1.0 / dim) - u * u
    return w * ((x - u) * lax.rsqrt(s + _EPS)) + b


def _tc_video_body(vf_ref, tt_ref, vidW_ref, tok_ref, pe_ref,
                   vcs, vc, vl2w, vl2b, out_ref):
    v = vf_ref[...]
    inv_v = 1.0 / v.shape[1]
    uv = jnp.sum(v, axis=-1, keepdims=True) * inv_v
    sv = jnp.sum(v * v, axis=-1, keepdims=True) * inv_v - uv * uv
    mv = jnp.dot(v.astype(jnp.bfloat16), vidW_ref[...],
                 preferred_element_type=jnp.float32)
    hv = (mv - uv * vcs[...]) * lax.rsqrt(sv + _EPS) + vc[...]
    hv = _ln(jnp.maximum(hv, 0.0), vl2w[...], vl2b[...], 768.0)
    ids = tt_ref[0]
    ids2 = ids.reshape(-1, 1)
    tt = jnp.where(ids2 == 0, tok_ref[0:1, :],
                   jnp.where(ids2 == 1, tok_ref[1:2, :], tok_ref[2:3, :]))
    out_ref[...] = (hv + tt + pe_ref[...]).astype(jnp.bfloat16)


def _tc_word_body(wet_ref, part_ref, wfcW_ref,
                  wcs, wc, wl2w, wl2b, flw, flb, out_ref):
    xt = wet_ref[...]
    inv = 1.0 / xt.shape[0]
    u_r = jnp.sum(xt, axis=0, keepdims=True) * inv
    s_r = jnp.sum(xt * xt, axis=0, keepdims=True) * inv - u_r * u_r
    u = u_r.reshape(-1, 1)
    s = s_r.reshape(-1, 1)
    mm = lax.dot_general(xt.astype(jnp.bfloat16), wfcW_ref[...],
                         (((0,), (0,)), ((), ())),
                         preferred_element_type=jnp.float32)
    hw = (mm - u * wcs[...]) * lax.rsqrt(s + _EPS) + wc[...]
    hw = _ln(jnp.maximum(hw, 0.0), wl2w[...], wl2b[...], 768.0)
    emb = hw + part_ref[...].astype(jnp.float32)
    out_ref[...] = _ln(emb, flw[...], flb[...], 768.0)


def _whole(i):
    return (0, 0)


def _tok_block(i):
    return (i, 0)


def _tc_video(vf, ttc, vidW, tok_table, pe_t, params):
    n, vid_feat = vf.shape
    hid = vidW.shape[1]
    in_specs = [
        pl.BlockSpec((_TB, vid_feat), _tok_block),
        pl.BlockSpec((1, 1, _TB), lambda i: (i, 0, 0)),
        pl.BlockSpec(vidW.shape, _whole),
        pl.BlockSpec(tok_table.shape, _whole),
        pl.BlockSpec(pe_t.shape, _whole),
    ] + [pl.BlockSpec(p.shape, _whole) for p in params]
    return pl.pallas_call(
        _tc_video_body,
        grid=(n // _TB,),
        in_specs=in_specs,
        out_specs=pl.BlockSpec((_TB, hid), _tok_block),
        out_shape=jax.ShapeDtypeStruct((n, hid), jnp.bfloat16),
    )(vf, ttc, vidW, tok_table, pe_t, *params)


def _tc_word(wet, partial, wfcW, params):
    wvec, n = wet.shape
    hid = wfcW.shape[1]
    in_specs = [
        pl.BlockSpec((wvec, _TB), lambda i: (0, i)),
        pl.BlockSpec((_TB, hid), _tok_block),
        pl.BlockSpec(wfcW.shape, _whole),
    ] + [pl.BlockSpec(p.shape, _whole) for p in params]
    return pl.pallas_call(
        _tc_word_body,
        grid=(n // _TB,),
        in_specs=in_specs,
        out_specs=pl.BlockSpec((_TB, hid), _tok_block),
        out_shape=jax.ShapeDtypeStruct((n, hid), jnp.float32),
    )(wet, partial, wfcW, *params)


def kernel(input_ids, video_features, token_type_ids, word_table, tok_table,
           wfc_ln1_w, wfc_ln1_b, wfc_W, wfc_b, wfc_ln2_w, wfc_ln2_b,
           vid_ln1_w, vid_ln1_b, vid_W, vid_b, vid_ln2_w, vid_ln2_b,
           final_ln_w, final_ln_b, pe):
    b, l = input_ids.shape
    n = b * l
    hid = vid_W.shape[1]

    ids = input_ids.reshape(n).astype(jnp.int32)
    wet = _sc_gather_t(word_table.T, ids)

    vf = video_features.reshape(n, video_features.shape[2])
    ttc = token_type_ids.astype(jnp.int32).reshape(n // _TB, 1, _TB)
    pe_t = jnp.tile(pe[:l], (_TB // l, 1))

    row = lambda p: p.reshape(1, -1)
    wfcWs = wfc_ln1_w[:, None] * wfc_W
    w_colsum = row(jnp.sum(wfcWs, axis=0))
    w_const = row(wfc_ln1_b @ wfc_W + wfc_b)
    vidWs = vid_ln1_w[:, None] * vid_W
    v_colsum = row(jnp.sum(vidWs, axis=0))
    v_const = row(vid_ln1_b @ vid_W + vid_b)

    partial = _tc_video(vf, ttc, vidWs.astype(jnp.bfloat16), tok_table, pe_t,
                        (v_colsum, v_const, row(vid_ln2_w), row(vid_ln2_b)))
    out = _tc_word(wet, partial, wfcWs.astype(jnp.bfloat16),
                   (w_colsum, w_const, row(wfc_ln2_w), row(wfc_ln2_b),
                    row(final_ln_w), row(final_ln_b)))
    return out.reshape(b, l, hid)
